# baseline (speedup 1.0000x reference)
import functools

import jax
import jax.numpy as jnp
from jax import lax
from jax.experimental import pallas as pl
from jax.experimental.pallas import tpu as pltpu
from jax.experimental.pallas import tpu_sc as plsc

F32 = jnp.float32


def _prep_body(x_ref, wea_ref, web_ref, be1_ref, w1a_ref, b1_ref,
               pa_ref, pb_ref, q_ref):
    xb = x_ref[...]
    pa_ref[...] = jnp.dot(xb, wea_ref[...], preferred_element_type=F32) + be1_ref[...]
    pb_ref[...] = jnp.dot(xb, web_ref[...], preferred_element_type=F32)
    q_ref[...] = jnp.dot(xb, w1a_ref[...], preferred_element_type=F32) + b1_ref[...]


def _edge_body(eb, d, ed, apa_ref, apb_ref, ea_ref, qr_ref, we1c_ref, we2_ref,
               be2_ref, w1b_ref, ean_ref, hn_ref):
    apa = apa_ref[...]
    apb = apb_ref[...]
    qr = qr_ref[...].reshape(eb, d)
    t = apa + apb + jnp.dot(ea_ref[...], we1c_ref[...],
                            preferred_element_type=F32)
    he = jax.nn.softplus(t)
    ean = jnp.dot(he, we2_ref[...], preferred_element_type=F32) + be2_ref[...]
    ean_ref[...] = ean
    u = qr + jnp.dot(ean, w1b_ref[...], preferred_element_type=F32)
    hn_ref[...] = jax.nn.softplus(u).astype(jnp.bfloat16).reshape(eb * d)


def _final_body(s_ref, c_ref, w2_ref, b2_ref, out_ref):
    sacc = s_ref[0].astype(F32) + s_ref[1].astype(F32)
    cnt = c_ref[0, :, 0:1] + c_ref[1, :, 0:1]
    out_ref[...] = (jnp.dot(sacc, w2_ref[...], preferred_element_type=F32)
                    + cnt * b2_ref[...])


def _make_gather(n, e, d, ed):
    g = 128
    ng = e // g
    nt = 32
    jmax = (ng + nt - 1) // nt
    mesh = plsc.VectorSubcoreMesh(core_axis_name="c", subcore_axis_name="s",
                                  num_cores=2, num_subcores=16)

    @functools.partial(
        pl.kernel,
        out_type=[jax.ShapeDtypeStruct((e, ed), F32),
                  jax.ShapeDtypeStruct((e, ed), F32),
                  jax.ShapeDtypeStruct((e, d), F32)],
        mesh=mesh,
        compiler_params=pltpu.CompilerParams(use_tc_tiling_on_sc=False),
        scratch_types=[pltpu.VMEM((g,), jnp.int32),
                       pltpu.VMEM((g,), jnp.int32),
                       pltpu.VMEM((g, ed), F32),
                       pltpu.VMEM((g, ed), F32),
                       pltpu.VMEM((g, d), F32),
                       pltpu.SemaphoreType.DMA,
                       pltpu.SemaphoreType.DMA,
                       pltpu.SemaphoreType.DMA],
    )
    def gather(pa_hbm, pb_hbm, q_hbm, row_hbm, col_hbm,
               apa_hbm, apb_hbm, qr_hbm, ir, ic, bpa, bpb, bq, s1, s2, s3):
        wid = lax.axis_index("s") * 2 + lax.axis_index("c")

        @pl.loop(0, jmax)
        def _(j):
            grp = wid + nt * j

            @pl.when(grp < ng)
            def _():
                base = grp * g
                pltpu.sync_copy(row_hbm.at[pl.ds(base, g)], ir)
                pltpu.sync_copy(col_hbm.at[pl.ds(base, g)], ic)
                ca = pltpu.async_copy(pa_hbm.at[ir], bpa, s1)
                cb = pltpu.async_copy(pb_hbm.at[ic], bpb, s2)
                cq = pltpu.async_copy(q_hbm.at[ir], bq, s3)
                ca.wait()
                cb.wait()
                cq.wait()
                pltpu.sync_copy(bpa, apa_hbm.at[pl.ds(base, g)])
                pltpu.sync_copy(bpb, apb_hbm.at[pl.ds(base, g)])
                pltpu.sync_copy(bq, qr_hbm.at[pl.ds(base, g)])

    return gather


def _make_scatter(n, e, d, ed):
    g = 128
    ng = e // g
    nt = 32
    jmax = (ng + nt - 1) // nt
    rpt = n // 16
    rb = rpt // 5
    mesh = plsc.VectorSubcoreMesh(core_axis_name="c", subcore_axis_name="s",
                                  num_cores=2, num_subcores=16)

    @functools.partial(
        pl.kernel,
        out_type=[jax.ShapeDtypeStruct((2 * n, d), jnp.bfloat16),
                  jax.ShapeDtypeStruct((2 * n, ed), F32)],
        mesh=mesh,
        compiler_params=pltpu.CompilerParams(use_tc_tiling_on_sc=False),
        scratch_types=[pltpu.VMEM((g,), jnp.int32),
                       pltpu.VMEM((g, d), jnp.bfloat16),
                       pltpu.VMEM((g, ed), F32),
                       pltpu.VMEM((rb, d), jnp.bfloat16),
                       pltpu.VMEM((rpt, ed), F32),
                       pltpu.VMEM_SHARED((n, d), jnp.bfloat16),
                       pltpu.VMEM_SHARED((n, ed), F32)],
    )
    def scatter(hn_hbm, col_hbm, s2_hbm, c2_hbm,
                ic, bh, ones, zb, cz, s_sh, c_sh):
        cid = lax.axis_index("c")
        sid = lax.axis_index("s")
        wid = sid * 2 + cid

        zvec = jnp.zeros((16,), F32)
        zvec16 = jnp.zeros((32,), jnp.bfloat16)
        onev = jnp.ones((16,), F32)

        @pl.loop(0, rb)
        def _(i):
            for k in range(d // 32):
                zb[i, pl.ds(k * 32, 32)] = zvec16

        @pl.loop(0, rpt)
        def _(i):
            cz[i, :] = zvec

        @pl.loop(0, g)
        def _(i):
            ones[i, :] = onev

        r0 = sid * rpt

        @pl.loop(0, 5)
        def _(k):
            pltpu.sync_copy(zb, s_sh.at[pl.ds(r0 + k * rb, rb)])

        pltpu.sync_copy(cz, c_sh.at[pl.ds(r0, rpt)])
        plsc.subcore_barrier()

        @pl.loop(0, jmax)
        def _(j):
            grp = wid + nt * j

            @pl.when(grp < ng)
            def _():
                base = grp * g
                pltpu.sync_copy(col_hbm.at[pl.ds(base, g)], ic)
                pltpu.sync_copy(hn_hbm.at[pl.ds(base, g)], bh)
                pltpu.sync_copy(bh, s_sh.at[ic], add=True)
                pltpu.sync_copy(ones, c_sh.at[ic], add=True)

        plsc.subcore_barrier()

        @pl.loop(0, 5)
        def _(k):
            r = r0 + k * rb
            pltpu.sync_copy(s_sh.at[pl.ds(r, rb)], zb)
            pltpu.sync_copy(zb, s2_hbm.at[pl.ds(cid * n + r, rb)])

        pltpu.sync_copy(c_sh.at[pl.ds(r0, rpt)], cz)
        pltpu.sync_copy(cz, c2_hbm.at[pl.ds(cid * n + r0, rpt)])

    return scatter


def kernel(x, edge_index, edge_attr, W1, b1, W2, b2, We1, be1, We2, be2):
    n, d = x.shape
    e, ed = edge_attr.shape
    row = edge_index[0]
    col = edge_index[1]

    we1a = We1[:d]
    we1b = We1[d:2 * d]
    we1c = We1[2 * d:]
    w1a = W1[:d]
    w1b = W1[d:]

    nb = 5
    bn = n // nb
    pa, pb, q = pl.pallas_call(
        _prep_body,
        grid=(nb,),
        in_specs=[pl.BlockSpec((bn, d), lambda i: (i, 0)),
                  pl.BlockSpec((d, ed), lambda i: (0, 0)),
                  pl.BlockSpec((d, ed), lambda i: (0, 0)),
                  pl.BlockSpec((1, ed), lambda i: (0, 0)),
                  pl.BlockSpec((d, d), lambda i: (0, 0)),
                  pl.BlockSpec((1, d), lambda i: (0, 0))],
        out_specs=[pl.BlockSpec((bn, ed), lambda i: (i, 0)),
                   pl.BlockSpec((bn, ed), lambda i: (i, 0)),
                   pl.BlockSpec((bn, d), lambda i: (i, 0))],
        out_shape=[jax.ShapeDtypeStruct((n, ed), F32),
                   jax.ShapeDtypeStruct((n, ed), F32),
                   jax.ShapeDtypeStruct((n, d), F32)],
    )(x, we1a, we1b, be1.reshape(1, ed), w1a, b1.reshape(1, d))

    apa, apb, qr = _make_gather(n, e, d, ed)(pa, pb, q, row, col)

    eb = 2560
    neb = e // eb
    ean, hn_flat = pl.pallas_call(
        functools.partial(_edge_body, eb, d, ed),
        grid=(neb,),
        in_specs=[pl.BlockSpec((eb, ed), lambda i: (i, 0)),
                  pl.BlockSpec((eb, ed), lambda i: (i, 0)),
                  pl.BlockSpec((eb, ed), lambda i: (i, 0)),
                  pl.BlockSpec((eb * d,), lambda i: (i,)),
                  pl.BlockSpec((ed, ed), lambda i: (0, 0)),
                  pl.BlockSpec((ed, ed), lambda i: (0, 0)),
                  pl.BlockSpec((1, ed), lambda i: (0, 0)),
                  pl.BlockSpec((ed, d), lambda i: (0, 0))],
        out_specs=[pl.BlockSpec((eb, ed), lambda i: (i, 0)),
                   pl.BlockSpec((eb * d,), lambda i: (i,))],
        out_shape=[jax.ShapeDtypeStruct((e, ed), F32),
                   jax.ShapeDtypeStruct((e * d,), jnp.bfloat16)],
    )(apa, apb, edge_attr, qr.reshape(e * d),
      we1c, We2, be2.reshape(1, ed), w1b)

    s2, c2 = _make_scatter(n, e, d, ed)(hn_flat.reshape(e, d), col)
    s2 = s2.reshape(2, n, d)
    c2 = c2.reshape(2, n, ed)

    x_new = pl.pallas_call(
        _final_body,
        grid=(nb,),
        in_specs=[pl.BlockSpec((2, bn, d), lambda i: (0, i, 0)),
                  pl.BlockSpec((2, bn, ed), lambda i: (0, i, 0)),
                  pl.BlockSpec((d, d), lambda i: (0, 0)),
                  pl.BlockSpec((1, d), lambda i: (0, 0))],
        out_specs=pl.BlockSpec((bn, d), lambda i: (i, 0)),
        out_shape=jax.ShapeDtypeStruct((n, d), F32),
    )(s2, c2, W2, b2.reshape(1, d))

    return (x_new, ean)


# speedup vs baseline: 1.0483x; 1.0483x over previous
import functools

import jax
import jax.numpy as jnp
from jax import lax
from jax.experimental import pallas as pl
from jax.experimental.pallas import tpu as pltpu
from jax.experimental.pallas import tpu_sc as plsc

F32 = jnp.float32
BF16 = jnp.bfloat16


def _prep_body(x_ref, wea_ref, web_ref, be1_ref, w1a_ref, b1_ref,
               pa_ref, pb_ref, q_ref):
    xb = x_ref[...]
    pa_ref[...] = jnp.dot(xb, wea_ref[...], preferred_element_type=F32) + be1_ref[...]
    pb_ref[...] = jnp.dot(xb, web_ref[...], preferred_element_type=F32)
    q = jnp.dot(xb, w1a_ref[...], preferred_element_type=F32) + b1_ref[...]
    q_ref[...] = q.astype(BF16)


def _edge_body(eb, d, ed, a_ref, ea_ref, qr_ref, we1c_ref, we2_ref,
               be2_ref, w1b_ref, ean_ref, hn_ref):
    qr = qr_ref[...].reshape(eb, d).astype(F32)
    t = a_ref[...] + jnp.dot(ea_ref[...], we1c_ref[...],
                             preferred_element_type=F32)
    he = jax.nn.softplus(t)
    ean = jnp.dot(he, we2_ref[...], preferred_element_type=F32) + be2_ref[...]
    ean_ref[...] = ean
    u = qr + jnp.dot(ean, w1b_ref[...], preferred_element_type=F32)
    hn_ref[...] = jax.nn.softplus(u).astype(BF16).reshape(eb * d)


def _final_body(s_ref, c_ref, w2_ref, b2_ref, out_ref):
    sacc = s_ref[0].astype(F32) + s_ref[1].astype(F32)
    cnt = lax.dot_general(c_ref[...], b2_ref[...], (((0,), (0,)), ((), ())),
                          preferred_element_type=F32)
    out_ref[...] = jnp.dot(sacc, w2_ref[...], preferred_element_type=F32) + cnt


def _make_gather(n, e, d, ed):
    g = 128
    ng = e // g
    nt = 32
    jmax = (ng + nt - 1) // nt
    mesh = plsc.VectorSubcoreMesh(core_axis_name="c", subcore_axis_name="s",
                                  num_cores=2, num_subcores=16)

    @functools.partial(
        pl.kernel,
        out_type=[jax.ShapeDtypeStruct((e, ed), F32),
                  jax.ShapeDtypeStruct((e, d), BF16)],
        mesh=mesh,
        compiler_params=pltpu.CompilerParams(use_tc_tiling_on_sc=False, needs_layout_passes=False),
        scratch_types=[pltpu.VMEM((2, g), jnp.int32),
                       pltpu.VMEM((2, g), jnp.int32),
                       pltpu.VMEM((2, g, ed), F32),
                       pltpu.VMEM((2, g, ed), F32),
                       pltpu.VMEM((2, g, ed), F32),
                       pltpu.VMEM((2, g, d), BF16),
                       pltpu.SemaphoreType.DMA,
                       pltpu.SemaphoreType.DMA,
                       pltpu.SemaphoreType.DMA,
                       pltpu.SemaphoreType.DMA],
    )
    def gather(pa_hbm, pb_hbm, q_hbm, row_hbm, col_hbm, a_hbm, qr_hbm,
               ir, ic, bpa, bpb, ba, bq, sidx, sp, sq, sout):
        wid = lax.axis_index("s") * 2 + lax.axis_index("c")

        def valid(j):
            return wid + nt * j < ng

        def issue_idx(j, b):
            @pl.when(valid(j))
            def _():
                base = (wid + nt * j) * g
                pltpu.async_copy(row_hbm.at[pl.ds(base, g)], ir.at[b], sidx)
                pltpu.async_copy(col_hbm.at[pl.ds(base, g)], ic.at[b], sidx)

        def wait_idx(j, b):
            @pl.when(valid(j))
            def _():
                base = (wid + nt * j) * g
                pltpu.make_async_copy(row_hbm.at[pl.ds(base, g)], ir.at[b],
                                      sidx).wait()
                pltpu.make_async_copy(col_hbm.at[pl.ds(base, g)], ic.at[b],
                                      sidx).wait()

        def issue_gathers(j, b):
            @pl.when(valid(j))
            def _():
                pltpu.async_copy(pa_hbm.at[ir.at[b]], bpa.at[b], sp)
                pltpu.async_copy(pb_hbm.at[ic.at[b]], bpb.at[b], sp)
                pltpu.async_copy(q_hbm.at[ir.at[b]], bq.at[b], sq)

        def drain_out(j, b):
            @pl.when(valid(j))
            def _():
                pltpu.make_async_copy(ba.at[b], a_hbm.at[pl.ds(0, g)],
                                      sout).wait()
                pltpu.make_async_copy(bq.at[b], qr_hbm.at[pl.ds(0, g)],
                                      sout).wait()

        issue_idx(0, 0)
        wait_idx(0, 0)
        issue_gathers(0, 0)

        @pl.loop(0, jmax)
        def _(j):
            b = lax.rem(j, 2)
            nb = 1 - b

            issue_idx(j + 1, nb)

            @pl.when(j >= 1)
            def _():
                drain_out(j - 1, nb)

            wait_idx(j + 1, nb)
            issue_gathers(j + 1, nb)

            @pl.when(valid(j))
            def _():
                base = (wid + nt * j) * g
                pltpu.make_async_copy(pa_hbm.at[ir.at[b]], bpa.at[b],
                                      sp).wait()
                pltpu.make_async_copy(pb_hbm.at[ic.at[b]], bpb.at[b],
                                      sp).wait()
                for i in range(g):
                    ba[b, i, :] = bpa[b, i, :] + bpb[b, i, :]
                pltpu.make_async_copy(q_hbm.at[ir.at[b]], bq.at[b], sq).wait()
                pltpu.async_copy(ba.at[b], a_hbm.at[pl.ds(base, g)], sout)
                pltpu.async_copy(bq.at[b], qr_hbm.at[pl.ds(base, g)], sout)

        drain_out(jmax - 1, (jmax - 1) % 2)

    return gather


def _make_scatter(n, e, d, ed):
    g = 128
    ng = e // g
    nt = 32
    jmax = (ng + nt - 1) // nt
    rpt = n // 16
    rb = rpt // 5
    mesh = plsc.VectorSubcoreMesh(core_axis_name="c", subcore_axis_name="s",
                                  num_cores=2, num_subcores=16)

    @functools.partial(
        pl.kernel,
        out_type=[jax.ShapeDtypeStruct((2 * n, d), BF16),
                  jax.ShapeDtypeStruct((nt, n), F32)],
        mesh=mesh,
        compiler_params=pltpu.CompilerParams(use_tc_tiling_on_sc=False, needs_layout_passes=False),
        scratch_types=[pltpu.VMEM((2, g), jnp.int32),
                       pltpu.VMEM((2, g, d), BF16),
                       pltpu.VMEM((n,), F32),
                       pltpu.VMEM((rb, d), BF16),
                       pltpu.VMEM_SHARED((n, d), BF16),
                       pltpu.SemaphoreType.DMA,
                       pltpu.SemaphoreType.DMA],
    )
    def scatter(hn_hbm, col_hbm, s2_hbm, c2_hbm,
                ic, bh, cl, zb, s_sh, sidx, shn):
        cid = lax.axis_index("c")
        sid = lax.axis_index("s")
        wid = sid * 2 + cid

        zvec = jnp.zeros((16,), F32)
        zvec16 = jnp.zeros((32,), BF16)
        onev = jnp.ones((16,), F32)

        @pl.loop(0, rb)
        def _(i):
            for k in range(d // 32):
                zb[i, pl.ds(k * 32, 32)] = zvec16

        @pl.loop(0, n // 16)
        def _(i):
            cl[pl.ds(i * 16, 16)] = zvec

        r0 = sid * rpt

        @pl.loop(0, rpt // rb)
        def _(k):
            pltpu.sync_copy(zb, s_sh.at[pl.ds(r0 + k * rb, rb)])

        plsc.subcore_barrier()

        def issue(j, b):
            grp = wid + nt * j

            @pl.when(grp < ng)
            def _():
                base = grp * g
                pltpu.async_copy(col_hbm.at[pl.ds(base, g)], ic.at[b], sidx)
                pltpu.async_copy(hn_hbm.at[pl.ds(base, g)], bh.at[b], shn)

        issue(0, 0)

        @pl.loop(0, jmax)
        def _(j):
            b = lax.rem(j, 2)
            grp = wid + nt * j
            issue(j + 1, 1 - b)

            @pl.when(grp < ng)
            def _():
                base = grp * g
                pltpu.make_async_copy(col_hbm.at[pl.ds(base, g)], ic.at[b],
                                      sidx).wait()
                pltpu.make_async_copy(hn_hbm.at[pl.ds(base, g)], bh.at[b],
                                      shn).wait()
                pltpu.sync_copy(bh.at[b], s_sh.at[ic.at[b]], add=True)
                for k in range(g // 16):
                    idxv = ic[b, pl.ds(k * 16, 16)]
                    plsc.addupdate_scatter(cl, [idxv], onev)

        plsc.subcore_barrier()

        @pl.loop(0, rpt // rb)
        def _(k):
            r = r0 + k * rb
            pltpu.sync_copy(s_sh.at[pl.ds(r, rb)], zb)
            pltpu.sync_copy(zb, s2_hbm.at[pl.ds(cid * n + r, rb)])

        pltpu.sync_copy(cl, c2_hbm.at[wid])

    return scatter


def kernel(x, edge_index, edge_attr, W1, b1, W2, b2, We1, be1, We2, be2):
    n, d = x.shape
    e, ed = edge_attr.shape
    nt = 32
    row = edge_index[0]
    col = edge_index[1]

    we1a = We1[:d]
    we1b = We1[d:2 * d]
    we1c = We1[2 * d:]
    w1a = W1[:d]
    w1b = W1[d:]

    nb = 5
    bn = n // nb
    pa, pb, q = pl.pallas_call(
        _prep_body,
        grid=(nb,),
        in_specs=[pl.BlockSpec((bn, d), lambda i: (i, 0)),
                  pl.BlockSpec((d, ed), lambda i: (0, 0)),
                  pl.BlockSpec((d, ed), lambda i: (0, 0)),
                  pl.BlockSpec((1, ed), lambda i: (0, 0)),
                  pl.BlockSpec((d, d), lambda i: (0, 0)),
                  pl.BlockSpec((1, d), lambda i: (0, 0))],
        out_specs=[pl.BlockSpec((bn, ed), lambda i: (i, 0)),
                   pl.BlockSpec((bn, ed), lambda i: (i, 0)),
                   pl.BlockSpec((bn, d), lambda i: (i, 0))],
        out_shape=[jax.ShapeDtypeStruct((n, ed), F32),
                   jax.ShapeDtypeStruct((n, ed), F32),
                   jax.ShapeDtypeStruct((n, d), BF16)],
    )(x, we1a, we1b, be1.reshape(1, ed), w1a, b1.reshape(1, d))

    a_sum, qr = _make_gather(n, e, d, ed)(pa, pb, q, row, col)

    eb = 2560
    neb = e // eb
    ean, hn_flat = pl.pallas_call(
        functools.partial(_edge_body, eb, d, ed),
        grid=(neb,),
        in_specs=[pl.BlockSpec((eb, ed), lambda i: (i, 0)),
                  pl.BlockSpec((eb, ed), lambda i: (i, 0)),
                  pl.BlockSpec((eb * d,), lambda i: (i,)),
                  pl.BlockSpec((ed, ed), lambda i: (0, 0)),
                  pl.BlockSpec((ed, ed), lambda i: (0, 0)),
                  pl.BlockSpec((1, ed), lambda i: (0, 0)),
                  pl.BlockSpec((ed, d), lambda i: (0, 0))],
        out_specs=[pl.BlockSpec((eb, ed), lambda i: (i, 0)),
                   pl.BlockSpec((eb * d,), lambda i: (i,))],
        out_shape=[jax.ShapeDtypeStruct((e, ed), F32),
                   jax.ShapeDtypeStruct((e * d,), BF16)],
    )(a_sum, edge_attr, qr.reshape(e * d),
      we1c, We2, be2.reshape(1, ed), w1b)

    s2, c2 = _make_scatter(n, e, d, ed)(hn_flat.reshape(e, d), col)
    s2 = s2.reshape(2, n, d)

    b2m = jnp.ones((nt, 1), F32) * b2.reshape(1, d)

    x_new = pl.pallas_call(
        _final_body,
        in_specs=[pl.BlockSpec((2, n, d), lambda: (0, 0, 0)),
                  pl.BlockSpec((nt, n), lambda: (0, 0)),
                  pl.BlockSpec((d, d), lambda: (0, 0)),
                  pl.BlockSpec((nt, d), lambda: (0, 0))],
        out_specs=pl.BlockSpec((n, d), lambda: (0, 0)),
        out_shape=jax.ShapeDtypeStruct((n, d), F32),
    )(s2, c2, W2, b2m)

    return (x_new, ean)


# speedup vs baseline: 1.6905x; 1.6126x over previous
import functools

import jax
import jax.numpy as jnp
from jax import lax
from jax.experimental import pallas as pl
from jax.experimental.pallas import tpu as pltpu
from jax.experimental.pallas import tpu_sc as plsc

F32 = jnp.float32
BF16 = jnp.bfloat16


def _prep_body(x_ref, wea_ref, web_ref, be1_ref, w1a_ref, b1_ref,
               pa_ref, pb_ref, q_ref):
    xb = x_ref[...]
    pa_ref[...] = jnp.dot(xb, wea_ref[...], preferred_element_type=F32) + be1_ref[...]
    pb_ref[...] = jnp.dot(xb, web_ref[...], preferred_element_type=F32)
    q = jnp.dot(xb, w1a_ref[...], preferred_element_type=F32) + b1_ref[...]
    q_ref[...] = q


def _edge_body(eb, d, ed, a_ref, ea_ref, qr_hbm, we1c_ref, we2_ref,
               be2_ref, w1b_ref, ean_ref, hn_ref, qrv, sem):
    i = pl.program_id(0)
    b = lax.rem(i, 2)

    def issue(step, buf):
        pltpu.make_async_copy(qr_hbm.at[pl.ds(step * eb, eb)], qrv.at[buf],
                              sem).start()

    @pl.when(i == 0)
    def _():
        issue(0, 0)

    @pl.when(i + 1 < pl.num_programs(0))
    def _():
        issue(i + 1, 1 - b)

    pltpu.make_async_copy(qr_hbm.at[pl.ds(i * eb, eb)], qrv.at[b], sem).wait()
    t = a_ref[...] + jnp.dot(ea_ref[...], we1c_ref[...],
                             preferred_element_type=F32)
    he = jax.nn.softplus(t)
    ean = jnp.dot(he, we2_ref[...], preferred_element_type=F32) + be2_ref[...]
    ean_ref[...] = ean
    u = qrv[b] + jnp.dot(ean, w1b_ref[...], preferred_element_type=F32)
    hn_ref[...] = jax.nn.softplus(u)


def _final_body(n, s_ref, c_ref, w2_ref, b2_ref, out_ref):
    sacc = s_ref[...]
    cnt = lax.dot_general(c_ref[...], b2_ref[...], (((0,), (0,)), ((), ())),
                          preferred_element_type=F32)
    out_ref[...] = jnp.dot(sacc, w2_ref[...], preferred_element_type=F32) + cnt


def _make_gather(n, e, d, ed):
    g = 128
    ng = e // g
    nt = 32
    jmax = (ng + nt - 1) // nt
    mesh = plsc.VectorSubcoreMesh(core_axis_name="c", subcore_axis_name="s",
                                  num_cores=2, num_subcores=16)

    @functools.partial(
        pl.kernel,
        out_type=[jax.ShapeDtypeStruct((e, ed), F32),
                  jax.ShapeDtypeStruct((e, d), F32)],
        mesh=mesh,
        compiler_params=pltpu.CompilerParams(use_tc_tiling_on_sc=False, needs_layout_passes=False),
        scratch_types=[pltpu.VMEM((2, g), jnp.int32),
                       pltpu.VMEM((2, g), jnp.int32),
                       pltpu.VMEM((2, g, ed), F32),
                       pltpu.VMEM((2, g, ed), F32),
                       pltpu.VMEM((2, g, ed), F32),
                       pltpu.VMEM((2, g, d), F32),
                       pltpu.SemaphoreType.DMA,
                       pltpu.SemaphoreType.DMA,
                       pltpu.SemaphoreType.DMA,
                       pltpu.SemaphoreType.DMA],
    )
    def gather(pa_hbm, pb_hbm, q_hbm, row_hbm, col_hbm, a_hbm, qr_hbm,
               ir, ic, bpa, bpb, ba, bq, sidx, sp, sq, sout):
        wid = lax.axis_index("s") * 2 + lax.axis_index("c")

        def valid(j):
            return wid + nt * j < ng

        def issue_idx(j, b):
            @pl.when(valid(j))
            def _():
                base = (wid + nt * j) * g
                pltpu.async_copy(row_hbm.at[pl.ds(base, g)], ir.at[b], sidx)
                pltpu.async_copy(col_hbm.at[pl.ds(base, g)], ic.at[b], sidx)

        def wait_idx(j, b):
            @pl.when(valid(j))
            def _():
                base = (wid + nt * j) * g
                pltpu.make_async_copy(row_hbm.at[pl.ds(base, g)], ir.at[b],
                                      sidx).wait()
                pltpu.make_async_copy(col_hbm.at[pl.ds(base, g)], ic.at[b],
                                      sidx).wait()

        def issue_gathers(j, b):
            @pl.when(valid(j))
            def _():
                pltpu.async_copy(pa_hbm.at[ir.at[b]], bpa.at[b], sp)
                pltpu.async_copy(pb_hbm.at[ic.at[b]], bpb.at[b], sp)
                pltpu.async_copy(q_hbm.at[ir.at[b]], bq.at[b], sq)

        def drain_out(j, b):
            @pl.when(valid(j))
            def _():
                pltpu.make_async_copy(ba.at[b], a_hbm.at[pl.ds(0, g)],
                                      sout).wait()
                pltpu.make_async_copy(bq.at[b], qr_hbm.at[pl.ds(0, g)],
                                      sout).wait()

        issue_idx(0, 0)
        wait_idx(0, 0)
        issue_gathers(0, 0)

        @pl.loop(0, jmax)
        def _(j):
            b = lax.rem(j, 2)
            nb = 1 - b

            issue_idx(j + 1, nb)

            @pl.when(j >= 1)
            def _():
                drain_out(j - 1, nb)

            wait_idx(j + 1, nb)
            issue_gathers(j + 1, nb)

            @pl.when(valid(j))
            def _():
                base = (wid + nt * j) * g
                pltpu.make_async_copy(pa_hbm.at[ir.at[b]], bpa.at[b],
                                      sp).wait()
                pltpu.make_async_copy(pb_hbm.at[ic.at[b]], bpb.at[b],
                                      sp).wait()
                for i in range(g):
                    ba[b, i, :] = bpa[b, i, :] + bpb[b, i, :]
                pltpu.make_async_copy(q_hbm.at[ir.at[b]], bq.at[b], sq).wait()
                pltpu.async_copy(ba.at[b], a_hbm.at[pl.ds(base, g)], sout)
                pltpu.async_copy(bq.at[b], qr_hbm.at[pl.ds(base, g)], sout)

        drain_out(jmax - 1, (jmax - 1) % 2)

    return gather


def _make_scatter(n, e, d, ed):
    g = 128
    ng = e // g
    nt = 32
    h = d // 2
    jmax = (ng + 16 - 1) // 16
    rpt = n // 16
    rb = rpt // 5
    mesh = plsc.VectorSubcoreMesh(core_axis_name="c", subcore_axis_name="s",
                                  num_cores=2, num_subcores=16)

    @functools.partial(
        pl.kernel,
        out_type=[jax.ShapeDtypeStruct((n, d), F32),
                  jax.ShapeDtypeStruct((nt, n), F32)],
        mesh=mesh,
        compiler_params=pltpu.CompilerParams(use_tc_tiling_on_sc=False, needs_layout_passes=False),
        scratch_types=[pltpu.VMEM((2, g), jnp.int32),
                       pltpu.VMEM((2, g, h), F32),
                       pltpu.VMEM((n,), F32),
                       pltpu.VMEM((rb, h), F32),
                       pltpu.VMEM_SHARED((n, h), F32),
                       pltpu.SemaphoreType.DMA,
                       pltpu.SemaphoreType.DMA],
    )
    def scatter(hn_hbm, col_hbm, s_hbm, c2_hbm,
                ic, bh, cl, zb, s_sh, sidx, shn):
        cid = lax.axis_index("c")
        sid = lax.axis_index("s")
        wid = sid * 2 + cid
        c0 = cid * h

        zvec = jnp.zeros((16,), F32)
        onev = jnp.ones((16,), F32)

        @pl.loop(0, rb)
        def _(i):
            for k in range(h // 16):
                zb[i, pl.ds(k * 16, 16)] = zvec

        @pl.loop(0, n // 16)
        def _(i):
            cl[pl.ds(i * 16, 16)] = zvec

        r0 = sid * rpt

        @pl.loop(0, rpt // rb)
        def _(k):
            pltpu.sync_copy(zb, s_sh.at[pl.ds(r0 + k * rb, rb)])

        plsc.subcore_barrier()

        def issue(j, b):
            grp = sid + 16 * j

            @pl.when(grp < ng)
            def _():
                base = grp * g
                pltpu.async_copy(col_hbm.at[pl.ds(base, g)], ic.at[b], sidx)
                pltpu.async_copy(hn_hbm.at[pl.ds(base, g), pl.ds(c0, h)],
                                 bh.at[b], shn)

        issue(0, 0)

        @pl.loop(0, jmax)
        def _(j):
            b = lax.rem(j, 2)
            grp = sid + 16 * j
            issue(j + 1, 1 - b)

            @pl.when(grp < ng)
            def _():
                base = grp * g
                pltpu.make_async_copy(col_hbm.at[pl.ds(base, g)], ic.at[b],
                                      sidx).wait()
                pltpu.make_async_copy(hn_hbm.at[pl.ds(base, g), pl.ds(c0, h)],
                                      bh.at[b], shn).wait()
                pltpu.sync_copy(bh.at[b], s_sh.at[ic.at[b]], add=True)

                @pl.when(cid == 0)
                def _():
                    for k in range(g // 16):
                        idxv = ic[b, pl.ds(k * 16, 16)]
                        plsc.addupdate_scatter(cl, [idxv], onev)

        plsc.subcore_barrier()

        @pl.loop(0, rpt // rb)
        def _(k):
            r = r0 + k * rb
            pltpu.sync_copy(s_sh.at[pl.ds(r, rb)], zb)
            pltpu.sync_copy(zb, s_hbm.at[pl.ds(r, rb), pl.ds(c0, h)])

        pltpu.sync_copy(cl, c2_hbm.at[wid])

    return scatter


def kernel(x, edge_index, edge_attr, W1, b1, W2, b2, We1, be1, We2, be2):
    n, d = x.shape
    e, ed = edge_attr.shape
    nt = 32
    row = edge_index[0]
    col = edge_index[1]

    we1a = We1[:d]
    we1b = We1[d:2 * d]
    we1c = We1[2 * d:]
    w1a = W1[:d]
    w1b = W1[d:]

    nb = 5
    bn = n // nb
    pa, pb, q = pl.pallas_call(
        _prep_body,
        grid=(nb,),
        in_specs=[pl.BlockSpec((bn, d), lambda i: (i, 0)),
                  pl.BlockSpec((d, ed), lambda i: (0, 0)),
                  pl.BlockSpec((d, ed), lambda i: (0, 0)),
                  pl.BlockSpec((1, ed), lambda i: (0, 0)),
                  pl.BlockSpec((d, d), lambda i: (0, 0)),
                  pl.BlockSpec((1, d), lambda i: (0, 0))],
        out_specs=[pl.BlockSpec((bn, ed), lambda i: (i, 0)),
                   pl.BlockSpec((bn, ed), lambda i: (i, 0)),
                   pl.BlockSpec((bn, d), lambda i: (i, 0))],
        out_shape=[jax.ShapeDtypeStruct((n, ed), F32),
                   jax.ShapeDtypeStruct((n, ed), F32),
                   jax.ShapeDtypeStruct((n, d), F32)],
    )(x, we1a, we1b, be1.reshape(1, ed), w1a, b1.reshape(1, d))

    a_sum, qr = _make_gather(n, e, d, ed)(pa, pb, q, row, col)

    eb = 2560
    neb = e // eb
    ean, hn = pl.pallas_call(
        functools.partial(_edge_body, eb, d, ed),
        grid=(neb,),
        in_specs=[pl.BlockSpec((eb, ed), lambda i: (i, 0)),
                  pl.BlockSpec((eb, ed), lambda i: (i, 0)),
                  pl.BlockSpec(memory_space=pl.ANY),
                  pl.BlockSpec((ed, ed), lambda i: (0, 0)),
                  pl.BlockSpec((ed, ed), lambda i: (0, 0)),
                  pl.BlockSpec((1, ed), lambda i: (0, 0)),
                  pl.BlockSpec((ed, d), lambda i: (0, 0))],
        out_specs=[pl.BlockSpec((eb, ed), lambda i: (i, 0)),
                   pl.BlockSpec((eb, d), lambda i: (i, 0))],
        out_shape=[jax.ShapeDtypeStruct((e, ed), F32),
                   jax.ShapeDtypeStruct((e, d), F32)],
        scratch_shapes=[pltpu.VMEM((2, eb, d), F32),
                        pltpu.SemaphoreType.DMA],
    )(a_sum, edge_attr, qr, we1c, We2, be2.reshape(1, ed), w1b)

    s_acc, c2 = _make_scatter(n, e, d, ed)(hn, col)

    b2m = jnp.ones((nt, 1), F32) * b2.reshape(1, d)

    x_new = pl.pallas_call(
        functools.partial(_final_body, n),
        in_specs=[pl.BlockSpec((n, d), lambda: (0, 0)),
                  pl.BlockSpec((nt, n), lambda: (0, 0)),
                  pl.BlockSpec((d, d), lambda: (0, 0)),
                  pl.BlockSpec((nt, d), lambda: (0, 0))],
        out_specs=pl.BlockSpec((n, d), lambda: (0, 0)),
        out_shape=jax.ShapeDtypeStruct((n, d), F32),
    )(s_acc, c2, W2, b2m)

    return (x_new, ean)


# speedup vs baseline: 2.0085x; 1.1881x over previous
import functools

import jax
import jax.numpy as jnp
from jax import lax
from jax.experimental import pallas as pl
from jax.experimental.pallas import tpu as pltpu
from jax.experimental.pallas import tpu_sc as plsc

F32 = jnp.float32
BF16 = jnp.bfloat16


def _prep_body(x_ref, wea_ref, web_ref, be1_ref, w1a_ref, b1_ref,
               pa_ref, pb_ref, q_ref):
    xb = x_ref[...]
    pa_ref[...] = jnp.dot(xb, wea_ref[...], preferred_element_type=F32) + be1_ref[...]
    pb_ref[...] = jnp.dot(xb, web_ref[...], preferred_element_type=F32)
    q = jnp.dot(xb, w1a_ref[...], preferred_element_type=F32) + b1_ref[...]
    q_ref[...] = q


def _edge_body(eb, d, ed, a_ref, ea_ref, qr_hbm, bd1_ref, bd2_ref,
               be2p_ref, bdw_ref, ean_ref, hn_ref, qrv, sem):
    i = pl.program_id(0)
    b = lax.rem(i, 2)

    def issue(step, buf):
        pltpu.make_async_copy(qr_hbm.at[pl.ds(step * eb, eb)], qrv.at[buf],
                              sem).start()

    @pl.when(i == 0)
    def _():
        issue(0, 0)

    @pl.when(i + 1 < pl.num_programs(0))
    def _():
        issue(i + 1, 1 - b)

    pltpu.make_async_copy(qr_hbm.at[pl.ds(i * eb, eb)], qrv.at[b], sem).wait()
    t_p = a_ref[...] + jnp.dot(ea_ref[...], bd1_ref[...],
                               preferred_element_type=F32)
    he_p = jax.nn.softplus(t_p)
    ean_p = jnp.dot(he_p, bd2_ref[...],
                    preferred_element_type=F32) + be2p_ref[...]
    ean_ref[...] = ean_p
    v = jnp.dot(ean_p, bdw_ref[...],
                preferred_element_type=F32).reshape(eb, d)
    u = qrv[b] + v
    hn_ref[...] = jax.nn.softplus(u)


def _final_body(n, s_ref, c_ref, w2_ref, b2_ref, out_ref):
    sacc = s_ref[...]
    cnt = lax.dot_general(c_ref[...], b2_ref[...], (((0,), (0,)), ((), ())),
                          preferred_element_type=F32)
    out_ref[...] = jnp.dot(sacc, w2_ref[...], preferred_element_type=F32) + cnt


def _make_gather(n, e, d, ed):
    g = 128
    ng = e // g
    nt = 32
    jmax = (ng + nt - 1) // nt
    mesh = plsc.VectorSubcoreMesh(core_axis_name="c", subcore_axis_name="s",
                                  num_cores=2, num_subcores=16)

    @functools.partial(
        pl.kernel,
        out_type=[jax.ShapeDtypeStruct((e * ed // 128, 128), F32),
                  jax.ShapeDtypeStruct((e, d), F32)],
        mesh=mesh,
        compiler_params=pltpu.CompilerParams(use_tc_tiling_on_sc=False, needs_layout_passes=False),
        scratch_types=[pltpu.VMEM((2, g), jnp.int32),
                       pltpu.VMEM((2, g), jnp.int32),
                       pltpu.VMEM((2, g, ed), F32),
                       pltpu.VMEM((2, g, ed), F32),
                       pltpu.VMEM((2, g * ed // 128, 128), F32),
                       pltpu.VMEM((2, g, d), F32),
                       pltpu.SemaphoreType.DMA,
                       pltpu.SemaphoreType.DMA,
                       pltpu.SemaphoreType.DMA,
                       pltpu.SemaphoreType.DMA],
    )
    def gather(pa_hbm, pb_hbm, q_hbm, row_hbm, col_hbm, a_hbm, qr_hbm,
               ir, ic, bpa, bpb, ba, bq, sidx, sp, sq, sout):
        wid = lax.axis_index("s") * 2 + lax.axis_index("c")

        def valid(j):
            return wid + nt * j < ng

        def issue_idx(j, b):
            @pl.when(valid(j))
            def _():
                base = (wid + nt * j) * g
                pltpu.async_copy(row_hbm.at[pl.ds(base, g)], ir.at[b], sidx)
                pltpu.async_copy(col_hbm.at[pl.ds(base, g)], ic.at[b], sidx)

        def wait_idx(j, b):
            @pl.when(valid(j))
            def _():
                base = (wid + nt * j) * g
                pltpu.make_async_copy(row_hbm.at[pl.ds(base, g)], ir.at[b],
                                      sidx).wait()
                pltpu.make_async_copy(col_hbm.at[pl.ds(base, g)], ic.at[b],
                                      sidx).wait()

        def issue_gathers(j, b):
            @pl.when(valid(j))
            def _():
                pltpu.async_copy(pa_hbm.at[ir.at[b]], bpa.at[b], sp)
                pltpu.async_copy(pb_hbm.at[ic.at[b]], bpb.at[b], sp)
                pltpu.async_copy(q_hbm.at[ir.at[b]], bq.at[b], sq)

        def drain_out(j, b):
            @pl.when(valid(j))
            def _():
                pltpu.make_async_copy(ba.at[b],
                                      a_hbm.at[pl.ds(0, g * ed // 128)],
                                      sout).wait()
                pltpu.make_async_copy(bq.at[b], qr_hbm.at[pl.ds(0, g)],
                                      sout).wait()

        issue_idx(0, 0)
        wait_idx(0, 0)
        issue_gathers(0, 0)

        @pl.loop(0, jmax)
        def _(j):
            b = lax.rem(j, 2)
            nb = 1 - b

            issue_idx(j + 1, nb)

            @pl.when(j >= 1)
            def _():
                drain_out(j - 1, nb)

            wait_idx(j + 1, nb)
            issue_gathers(j + 1, nb)

            @pl.when(valid(j))
            def _():
                base = (wid + nt * j) * g
                pltpu.make_async_copy(pa_hbm.at[ir.at[b]], bpa.at[b],
                                      sp).wait()
                pltpu.make_async_copy(pb_hbm.at[ic.at[b]], bpb.at[b],
                                      sp).wait()
                for i in range(g):
                    ba[b, i // 8, pl.ds((i % 8) * ed, ed)] = (
                        bpa[b, i, :] + bpb[b, i, :])
                pltpu.make_async_copy(q_hbm.at[ir.at[b]], bq.at[b], sq).wait()
                pltpu.async_copy(
                    ba.at[b],
                    a_hbm.at[pl.ds((wid + nt * j) * (g * ed // 128),
                                   g * ed // 128)], sout)
                pltpu.async_copy(bq.at[b], qr_hbm.at[pl.ds(base, g)], sout)

        drain_out(jmax - 1, (jmax - 1) % 2)

    return gather


def _make_scatter(n, e, d, ed):
    g = 128
    ng = e // g
    nt = 32
    h = d // 2
    jmax = (ng + 16 - 1) // 16
    rpt = n // 16
    rb = rpt // 5
    mesh = plsc.VectorSubcoreMesh(core_axis_name="c", subcore_axis_name="s",
                                  num_cores=2, num_subcores=16)

    @functools.partial(
        pl.kernel,
        out_type=[jax.ShapeDtypeStruct((n, d), F32),
                  jax.ShapeDtypeStruct((nt, n), F32)],
        mesh=mesh,
        compiler_params=pltpu.CompilerParams(use_tc_tiling_on_sc=False, needs_layout_passes=False),
        scratch_types=[pltpu.VMEM((2, g), jnp.int32),
                       pltpu.VMEM((2, g, h), F32),
                       pltpu.VMEM((n,), F32),
                       pltpu.VMEM((rb, h), F32),
                       pltpu.VMEM_SHARED((n, h), F32),
                       pltpu.SemaphoreType.DMA,
                       pltpu.SemaphoreType.DMA],
    )
    def scatter(hn_hbm, col_hbm, s_hbm, c2_hbm,
                ic, bh, cl, zb, s_sh, sidx, shn):
        cid = lax.axis_index("c")
        sid = lax.axis_index("s")
        wid = sid * 2 + cid
        c0 = cid * h

        zvec = jnp.zeros((16,), F32)
        onev = jnp.ones((16,), F32)

        @pl.loop(0, rb)
        def _(i):
            for k in range(h // 16):
                zb[i, pl.ds(k * 16, 16)] = zvec

        @pl.loop(0, n // 16)
        def _(i):
            cl[pl.ds(i * 16, 16)] = zvec

        r0 = sid * rpt

        @pl.loop(0, rpt // rb)
        def _(k):
            pltpu.sync_copy(zb, s_sh.at[pl.ds(r0 + k * rb, rb)])

        plsc.subcore_barrier()

        def issue(j, b):
            grp = sid + 16 * j

            @pl.when(grp < ng)
            def _():
                base = grp * g
                pltpu.async_copy(col_hbm.at[pl.ds(base, g)], ic.at[b], sidx)
                pltpu.async_copy(hn_hbm.at[pl.ds(base, g), pl.ds(c0, h)],
                                 bh.at[b], shn)

        issue(0, 0)

        @pl.loop(0, jmax)
        def _(j):
            b = lax.rem(j, 2)
            grp = sid + 16 * j
            issue(j + 1, 1 - b)

            @pl.when(grp < ng)
            def _():
                base = grp * g
                pltpu.make_async_copy(col_hbm.at[pl.ds(base, g)], ic.at[b],
                                      sidx).wait()
                pltpu.make_async_copy(hn_hbm.at[pl.ds(base, g), pl.ds(c0, h)],
                                      bh.at[b], shn).wait()
                pltpu.sync_copy(bh.at[b], s_sh.at[ic.at[b]], add=True)

                @pl.when(cid == 0)
                def _():
                    for k in range(g // 16):
                        idxv = ic[b, pl.ds(k * 16, 16)]
                        plsc.addupdate_scatter(cl, [idxv], onev)

        plsc.subcore_barrier()

        @pl.loop(0, rpt // rb)
        def _(k):
            r = r0 + k * rb
            pltpu.sync_copy(s_sh.at[pl.ds(r, rb)], zb)
            pltpu.sync_copy(zb, s_hbm.at[pl.ds(r, rb), pl.ds(c0, h)])

        pltpu.sync_copy(cl, c2_hbm.at[wid])

    return scatter


def kernel(x, edge_index, edge_attr, W1, b1, W2, b2, We1, be1, We2, be2):
    n, d = x.shape
    e, ed = edge_attr.shape
    nt = 32
    row = edge_index[0]
    col = edge_index[1]

    we1a = We1[:d]
    we1b = We1[d:2 * d]
    we1c = We1[2 * d:]
    w1a = W1[:d]
    w1b = W1[d:]

    nb = 5
    bn = n // nb
    pa, pb, q = pl.pallas_call(
        _prep_body,
        grid=(nb,),
        in_specs=[pl.BlockSpec((bn, d), lambda i: (i, 0)),
                  pl.BlockSpec((d, ed), lambda i: (0, 0)),
                  pl.BlockSpec((d, ed), lambda i: (0, 0)),
                  pl.BlockSpec((1, ed), lambda i: (0, 0)),
                  pl.BlockSpec((d, d), lambda i: (0, 0)),
                  pl.BlockSpec((1, d), lambda i: (0, 0))],
        out_specs=[pl.BlockSpec((bn, ed), lambda i: (i, 0)),
                   pl.BlockSpec((bn, ed), lambda i: (i, 0)),
                   pl.BlockSpec((bn, d), lambda i: (i, 0))],
        out_shape=[jax.ShapeDtypeStruct((n, ed), F32),
                   jax.ShapeDtypeStruct((n, ed), F32),
                   jax.ShapeDtypeStruct((n, d), F32)],
    )(x, we1a, we1b, be1.reshape(1, ed), w1a, b1.reshape(1, d))

    a_sum, qr = _make_gather(n, e, d, ed)(pa, pb, q, row, col)

    eb = 2560
    ebp = eb * ed // 128
    neb = e // eb
    ep = e * ed // 128

    eye8 = jnp.eye(8, dtype=F32)
    bd1 = jnp.einsum("pq,kj->pkqj", eye8, we1c).reshape(8 * ed, 8 * ed)
    bd2 = jnp.einsum("pq,kj->pkqj", eye8, We2).reshape(8 * ed, 8 * ed)
    bdw = jnp.einsum("pq,kj->pkqj", eye8, w1b).reshape(8 * ed, 8 * d)
    be2p = jnp.tile(be2, 8).reshape(1, 8 * ed)
    ea_p = edge_attr.reshape(ep, 128)

    ean_p, hn = pl.pallas_call(
        functools.partial(_edge_body, eb, d, ed),
        grid=(neb,),
        in_specs=[pl.BlockSpec((ebp, 128), lambda i: (i, 0)),
                  pl.BlockSpec((ebp, 128), lambda i: (i, 0)),
                  pl.BlockSpec(memory_space=pl.ANY),
                  pl.BlockSpec((8 * ed, 8 * ed), lambda i: (0, 0)),
                  pl.BlockSpec((8 * ed, 8 * ed), lambda i: (0, 0)),
                  pl.BlockSpec((1, 8 * ed), lambda i: (0, 0)),
                  pl.BlockSpec((8 * ed, 8 * d), lambda i: (0, 0))],
        out_specs=[pl.BlockSpec((ebp, 128), lambda i: (i, 0)),
                   pl.BlockSpec((eb, d), lambda i: (i, 0))],
        out_shape=[jax.ShapeDtypeStruct((ep, 128), F32),
                   jax.ShapeDtypeStruct((e, d), F32)],
        scratch_shapes=[pltpu.VMEM((2, eb, d), F32),
                        pltpu.SemaphoreType.DMA],
    )(a_sum, ea_p, qr, bd1, bd2, be2p, bdw)
    ean = ean_p.reshape(e, ed)

    s_acc, c2 = _make_scatter(n, e, d, ed)(hn, col)

    b2m = jnp.ones((nt, 1), F32) * b2.reshape(1, d)

    x_new = pl.pallas_call(
        functools.partial(_final_body, n),
        in_specs=[pl.BlockSpec((n, d), lambda: (0, 0)),
                  pl.BlockSpec((nt, n), lambda: (0, 0)),
                  pl.BlockSpec((d, d), lambda: (0, 0)),
                  pl.BlockSpec((nt, d), lambda: (0, 0))],
        out_specs=pl.BlockSpec((n, d), lambda: (0, 0)),
        out_shape=jax.ShapeDtypeStruct((n, d), F32),
    )(s_acc, c2, W2, b2m)

    return (x_new, ean)


# speedup vs baseline: 2.1441x; 1.0675x over previous
import functools

import jax
import jax.numpy as jnp
from jax import lax
from jax.experimental import pallas as pl
from jax.experimental.pallas import tpu as pltpu
from jax.experimental.pallas import tpu_sc as plsc

F32 = jnp.float32
BF16 = jnp.bfloat16


def _prep_body(x_ref, wea_ref, web_ref, be1_ref, w1a_ref, b1_ref,
               pa_ref, pb_ref, q_ref):
    xb = x_ref[...]
    pa_ref[...] = jnp.dot(xb, wea_ref[...], preferred_element_type=F32) + be1_ref[...]
    pb_ref[...] = jnp.dot(xb, web_ref[...], preferred_element_type=F32)
    q = jnp.dot(xb, w1a_ref[...], preferred_element_type=F32) + b1_ref[...]
    q_ref[...] = q


def _edge_body(eb, d, ed, a_ref, ea_ref, qr_hbm, bd1_ref, bd2_ref,
               be2p_ref, bdw_ref, ean_ref, hn_ref, qrv, sem):
    i = pl.program_id(0)
    b = lax.rem(i, 2)

    def issue(step, buf):
        pltpu.make_async_copy(qr_hbm.at[pl.ds(step * eb, eb)], qrv.at[buf],
                              sem).start()

    @pl.when(i == 0)
    def _():
        issue(0, 0)

    @pl.when(i + 1 < pl.num_programs(0))
    def _():
        issue(i + 1, 1 - b)

    pltpu.make_async_copy(qr_hbm.at[pl.ds(i * eb, eb)], qrv.at[b], sem).wait()
    t_p = a_ref[...] + jnp.dot(ea_ref[...], bd1_ref[...],
                               preferred_element_type=F32)
    he_p = jax.nn.softplus(t_p)
    ean_p = jnp.dot(he_p, bd2_ref[...],
                    preferred_element_type=F32) + be2p_ref[...]
    ean_ref[...] = ean_p
    v = jnp.dot(ean_p, bdw_ref[...],
                preferred_element_type=F32).reshape(eb, d)
    u = qrv[b] + v
    hn_ref[...] = jax.nn.softplus(u)


def _final_body(n, s_ref, c_ref, w2_ref, b2_ref, out_ref):
    sacc = s_ref[...]
    cnt = lax.dot_general(c_ref[...], b2_ref[...], (((0,), (0,)), ((), ())),
                          preferred_element_type=F32)
    out_ref[...] = jnp.dot(sacc, w2_ref[...], preferred_element_type=F32) + cnt


def _make_gather(n, e, d, ed):
    g = 128
    ng = e // g
    nt = 32
    jmax = (ng + nt - 1) // nt
    mesh = plsc.VectorSubcoreMesh(core_axis_name="c", subcore_axis_name="s",
                                  num_cores=2, num_subcores=16)

    @functools.partial(
        pl.kernel,
        out_type=[jax.ShapeDtypeStruct((e * ed // 128, 128), F32),
                  jax.ShapeDtypeStruct((e, d), F32)],
        mesh=mesh,
        compiler_params=pltpu.CompilerParams(use_tc_tiling_on_sc=False, needs_layout_passes=False),
        scratch_types=[pltpu.VMEM((2, g), jnp.int32),
                       pltpu.VMEM((2, g), jnp.int32),
                       pltpu.VMEM((2, g, ed), F32),
                       pltpu.VMEM((2, g, ed), F32),
                       pltpu.VMEM((2, g * ed // 128, 128), F32),
                       pltpu.VMEM((2, g, d), F32),
                       pltpu.SemaphoreType.DMA,
                       pltpu.SemaphoreType.DMA,
                       pltpu.SemaphoreType.DMA,
                       pltpu.SemaphoreType.DMA],
    )
    def gather(pa_hbm, pb_hbm, q_hbm, row_hbm, col_hbm, a_hbm, qr_hbm,
               ir, ic, bpa, bpb, ba, bq, sidx, sp, sq, sout):
        wid = lax.axis_index("s") * 2 + lax.axis_index("c")

        def valid(j):
            return wid + nt * j < ng

        def issue_idx(j, b):
            @pl.when(valid(j))
            def _():
                base = (wid + nt * j) * g
                pltpu.async_copy(row_hbm.at[pl.ds(base, g)], ir.at[b], sidx)
                pltpu.async_copy(col_hbm.at[pl.ds(base, g)], ic.at[b], sidx)

        def wait_idx(j, b):
            @pl.when(valid(j))
            def _():
                base = (wid + nt * j) * g
                pltpu.make_async_copy(row_hbm.at[pl.ds(base, g)], ir.at[b],
                                      sidx).wait()
                pltpu.make_async_copy(col_hbm.at[pl.ds(base, g)], ic.at[b],
                                      sidx).wait()

        def issue_gathers(j, b):
            @pl.when(valid(j))
            def _():
                pltpu.async_copy(pa_hbm.at[ir.at[b]], bpa.at[b], sp)
                pltpu.async_copy(pb_hbm.at[ic.at[b]], bpb.at[b], sp)
                pltpu.async_copy(q_hbm.at[ir.at[b]], bq.at[b], sq)

        def drain_out(j, b):
            @pl.when(valid(j))
            def _():
                pltpu.make_async_copy(ba.at[b],
                                      a_hbm.at[pl.ds(0, g * ed // 128)],
                                      sout).wait()
                pltpu.make_async_copy(bq.at[b], qr_hbm.at[pl.ds(0, g)],
                                      sout).wait()

        issue_idx(0, 0)
        wait_idx(0, 0)
        issue_gathers(0, 0)

        @pl.loop(0, jmax)
        def _(j):
            b = lax.rem(j, 2)
            nb = 1 - b

            issue_idx(j + 1, nb)

            @pl.when(j >= 1)
            def _():
                drain_out(j - 1, nb)

            wait_idx(j + 1, nb)
            issue_gathers(j + 1, nb)

            @pl.when(valid(j))
            def _():
                base = (wid + nt * j) * g
                pltpu.make_async_copy(pa_hbm.at[ir.at[b]], bpa.at[b],
                                      sp).wait()
                pltpu.make_async_copy(pb_hbm.at[ic.at[b]], bpb.at[b],
                                      sp).wait()
                for i in range(g):
                    ba[b, i // 8, pl.ds((i % 8) * ed, ed)] = (
                        bpa[b, i, :] + bpb[b, i, :])
                pltpu.make_async_copy(q_hbm.at[ir.at[b]], bq.at[b], sq).wait()
                pltpu.async_copy(
                    ba.at[b],
                    a_hbm.at[pl.ds((wid + nt * j) * (g * ed // 128),
                                   g * ed // 128)], sout)
                pltpu.async_copy(bq.at[b], qr_hbm.at[pl.ds(base, g)], sout)

        drain_out(jmax - 1, (jmax - 1) % 2)

    return gather


def _make_scatter(n, e, d, ed):
    g = 128
    ng = e // g
    nt = 32
    h = d // 2
    jmax = (ng + 16 - 1) // 16
    rpt = n // 16
    rb = rpt // 5
    mesh = plsc.VectorSubcoreMesh(core_axis_name="c", subcore_axis_name="s",
                                  num_cores=2, num_subcores=16)

    @functools.partial(
        pl.kernel,
        out_type=[jax.ShapeDtypeStruct((n, d), F32),
                  jax.ShapeDtypeStruct((nt, n), F32)],
        mesh=mesh,
        compiler_params=pltpu.CompilerParams(use_tc_tiling_on_sc=False, needs_layout_passes=False),
        scratch_types=[pltpu.VMEM((2, g), jnp.int32),
                       pltpu.VMEM((2, g, h), F32),
                       pltpu.VMEM((n,), F32),
                       pltpu.VMEM((rb, h), F32),
                       pltpu.VMEM_SHARED((n, h), F32),
                       pltpu.SemaphoreType.DMA,
                       pltpu.SemaphoreType.DMA],
    )
    def scatter(hn_hbm, col_hbm, s_hbm, c2_hbm,
                ic, bh, cl, zb, s_sh, sidx, shn):
        cid = lax.axis_index("c")
        sid = lax.axis_index("s")
        wid = sid * 2 + cid
        c0 = cid * h

        zvec = jnp.zeros((16,), F32)
        onev = jnp.ones((16,), F32)

        @pl.loop(0, rb)
        def _(i):
            for k in range(h // 16):
                zb[i, pl.ds(k * 16, 16)] = zvec

        @pl.loop(0, n // 16)
        def _(i):
            cl[pl.ds(i * 16, 16)] = zvec

        r0 = sid * rpt

        @pl.loop(0, rpt // rb)
        def _(k):
            pltpu.sync_copy(zb, s_sh.at[pl.ds(r0 + k * rb, rb)])

        plsc.subcore_barrier()

        def issue(j, b):
            grp = sid + 16 * j

            @pl.when(grp < ng)
            def _():
                base = grp * g
                pltpu.async_copy(col_hbm.at[pl.ds(base, g)], ic.at[b], sidx)
                pltpu.async_copy(hn_hbm.at[pl.ds(base, g), pl.ds(c0, h)],
                                 bh.at[b], shn)

        issue(0, 0)

        @pl.loop(0, jmax)
        def _(j):
            b = lax.rem(j, 2)
            grp = sid + 16 * j
            issue(j + 1, 1 - b)

            @pl.when(grp < ng)
            def _():
                base = grp * g
                pltpu.make_async_copy(col_hbm.at[pl.ds(base, g)], ic.at[b],
                                      sidx).wait()
                pltpu.make_async_copy(hn_hbm.at[pl.ds(base, g), pl.ds(c0, h)],
                                      bh.at[b], shn).wait()
                pltpu.sync_copy(bh.at[b], s_sh.at[ic.at[b]], add=True)

                @pl.when(cid == 0)
                def _():
                    for k in range(g // 16):
                        idxv = ic[b, pl.ds(k * 16, 16)]
                        plsc.addupdate_scatter(cl, [idxv], onev)

        plsc.subcore_barrier()

        @pl.loop(0, rpt // rb)
        def _(k):
            r = r0 + k * rb
            pltpu.sync_copy(s_sh.at[pl.ds(r, rb)], zb)
            pltpu.sync_copy(zb, s_hbm.at[pl.ds(r, rb), pl.ds(c0, h)])

        pltpu.sync_copy(cl, c2_hbm.at[wid])

    return scatter


def kernel(x, edge_index, edge_attr, W1, b1, W2, b2, We1, be1, We2, be2):
    n, d = x.shape
    e, ed = edge_attr.shape
    nt = 32
    row = edge_index[0]
    col = edge_index[1]

    we1a = We1[:d]
    we1b = We1[d:2 * d]
    we1c = We1[2 * d:]
    w1a = W1[:d]
    w1b = W1[d:]

    nb = 5
    bn = n // nb
    pa, pb, q = pl.pallas_call(
        _prep_body,
        grid=(nb,),
        in_specs=[pl.BlockSpec((bn, d), lambda i: (i, 0)),
                  pl.BlockSpec((d, ed), lambda i: (0, 0)),
                  pl.BlockSpec((d, ed), lambda i: (0, 0)),
                  pl.BlockSpec((1, ed), lambda i: (0, 0)),
                  pl.BlockSpec((d, d), lambda i: (0, 0)),
                  pl.BlockSpec((1, d), lambda i: (0, 0))],
        out_specs=[pl.BlockSpec((bn, ed), lambda i: (i, 0)),
                   pl.BlockSpec((bn, ed), lambda i: (i, 0)),
                   pl.BlockSpec((bn, d), lambda i: (i, 0))],
        out_shape=[jax.ShapeDtypeStruct((n, ed), F32),
                   jax.ShapeDtypeStruct((n, ed), F32),
                   jax.ShapeDtypeStruct((n, d), F32)],
    )(x, we1a, we1b, be1.reshape(1, ed), w1a, b1.reshape(1, d))

    a_sum, qr = _make_gather(n, e, d, ed)(pa, pb, q, row, col)

    eb = 5120
    ebp = eb * ed // 128
    neb = e // eb
    ep = e * ed // 128

    eye8 = jnp.eye(8, dtype=F32)
    bd1 = jnp.einsum("pq,kj->pkqj", eye8, we1c).reshape(8 * ed, 8 * ed)
    bd2 = jnp.einsum("pq,kj->pkqj", eye8, We2).reshape(8 * ed, 8 * ed)
    bdw = jnp.einsum("pq,kj->pkqj", eye8, w1b).reshape(8 * ed, 8 * d)
    be2p = jnp.tile(be2, 8).reshape(1, 8 * ed)
    ea_p = edge_attr.reshape(ep, 128)

    ean_p, hn = pl.pallas_call(
        functools.partial(_edge_body, eb, d, ed),
        grid=(neb,),
        in_specs=[pl.BlockSpec((ebp, 128), lambda i: (i, 0)),
                  pl.BlockSpec((ebp, 128), lambda i: (i, 0)),
                  pl.BlockSpec(memory_space=pl.ANY),
                  pl.BlockSpec((8 * ed, 8 * ed), lambda i: (0, 0)),
                  pl.BlockSpec((8 * ed, 8 * ed), lambda i: (0, 0)),
                  pl.BlockSpec((1, 8 * ed), lambda i: (0, 0)),
                  pl.BlockSpec((8 * ed, 8 * d), lambda i: (0, 0))],
        out_specs=[pl.BlockSpec((ebp, 128), lambda i: (i, 0)),
                   pl.BlockSpec((eb, d), lambda i: (i, 0))],
        out_shape=[jax.ShapeDtypeStruct((ep, 128), F32),
                   jax.ShapeDtypeStruct((e, d), F32)],
        scratch_shapes=[pltpu.VMEM((2, eb, d), F32),
                        pltpu.SemaphoreType.DMA],
    )(a_sum, ea_p, qr, bd1, bd2, be2p, bdw)
    ean = ean_p.reshape(e, ed)

    s_acc, c2 = _make_scatter(n, e, d, ed)(hn, col)

    b2m = jnp.ones((nt, 1), F32) * b2.reshape(1, d)

    x_new = pl.pallas_call(
        functools.partial(_final_body, n),
        in_specs=[pl.BlockSpec((n, d), lambda: (0, 0)),
                  pl.BlockSpec((nt, n), lambda: (0, 0)),
                  pl.BlockSpec((d, d), lambda: (0, 0)),
                  pl.BlockSpec((nt, d), lambda: (0, 0))],
        out_specs=pl.BlockSpec((n, d), lambda: (0, 0)),
        out_shape=jax.ShapeDtypeStruct((n, d), F32),
    )(s_acc, c2, W2, b2m)

    return (x_new, ean)
